# baseline (speedup 1.0000x reference)
import jax
import jax.numpy as jnp
from jax import lax
from jax.experimental import pallas as pl
from jax.experimental.pallas import tpu as pltpu
from jax.experimental.pallas import tpu_sc as plsc

N = 10000
E = 320000
D = 128
HD = D // 2
NS = 16
NC = 2
CH = 128
EC = 64
NCHUNK = -(-E // (NS * EC))
EP = NCHUNK * NS * EC
EPAD = EP - E
NTAB = N + 16
TPT = NTAB // NS
NPT = 640
NPADTOT = NPT * NS
LAST_R0 = (N // CH) * CH
LAST_SZ = N - LAST_R0


def _tile_body(author_hbm, srcp_hbm, dstp_hbm, out_hbm,
               srcv, dstv, rows, ones, zbuf, degv, tab, acc, deg,
               gsem, dsem):
    h = lax.axis_index("c")
    s = lax.axis_index("s")

    def fill_rows(i, _):
        for k in range(HD // 16):
            rows[i, pl.ds(k * 16, 16)] = jnp.zeros((16,), jnp.float32)
        return 0
    lax.fori_loop(0, CH, fill_rows, 0)

    def fill_1d(i, _):
        zbuf[pl.ds(i * 16, 16)] = jnp.zeros((16,), jnp.float32)
        ones[pl.ds(i * 16, 16)] = jnp.ones((16,), jnp.float32)
        return 0
    lax.fori_loop(0, CH // 16, fill_1d, 0)

    def fill_z(i, _):
        zbuf[pl.ds(CH + i * 16, 16)] = jnp.zeros((16,), jnp.float32)
        return 0
    lax.fori_loop(0, (NPT - CH) // 16, fill_z, 0)

    n0 = s * NPT
    for c in range(NPT // CH):
        pltpu.sync_copy(rows, acc.at[pl.ds(n0 + c * CH, CH)])
    pltpu.sync_copy(zbuf, deg.at[pl.ds(n0, NPT)])
    pltpu.sync_copy(author_hbm.at[h, pl.ds(s * TPT, TPT)],
                    tab.at[pl.ds(s * TPT, TPT)])

    pltpu.sync_copy(srcp_hbm.at[s], srcv)
    pltpu.sync_copy(dstp_hbm.at[s], dstv)

    plsc.subcore_barrier()

    def pair(i, _):
        ja = 2 * i
        jb = 2 * i + 1
        da = pltpu.async_copy(tab.at[srcv.at[ja]],
                              rows.at[pl.ds(0, EC)], gsem)
        db = pltpu.async_copy(tab.at[srcv.at[jb]],
                              rows.at[pl.ds(EC, EC)], gsem)
        da.wait()
        db.wait()
        pltpu.async_copy(ones.at[pl.ds(0, EC)], deg.at[dstv.at[ja]], dsem, add=True)
        pltpu.async_copy(ones.at[pl.ds(0, EC)], deg.at[dstv.at[jb]], dsem, add=True)
        pltpu.sync_copy(rows.at[pl.ds(0, EC)], acc.at[dstv.at[ja]],
                        add=True)
        pltpu.sync_copy(rows.at[pl.ds(EC, EC)], acc.at[dstv.at[jb]],
                        add=True)
        return 0
    lax.fori_loop(0, NCHUNK // 2, pair, 0)

    jl = NCHUNK - 1
    pltpu.async_copy(tab.at[srcv.at[jl]], rows.at[pl.ds(0, EC)],
                     gsem).wait()
    pltpu.async_copy(ones.at[pl.ds(0, EC)], deg.at[dstv.at[jl]], dsem, add=True)
    pltpu.sync_copy(rows.at[pl.ds(0, EC)], acc.at[dstv.at[jl]], add=True)

    def drain(j, _):
        pltpu.make_async_copy(ones.at[pl.ds(0, EC)], deg.at[dstv.at[j]], dsem).wait()
        return 0
    lax.fori_loop(0, NCHUNK, drain, 0)

    plsc.subcore_barrier()

    def fin_chunk(r0, nrows):
        pltpu.sync_copy(acc.at[pl.ds(r0, nrows)], rows.at[pl.ds(0, nrows)])
        pltpu.sync_copy(deg.at[pl.ds(r0, nrows)], degv.at[pl.ds(0, nrows)])

        @pl.when(r0 == 0)
        def _():
            v = degv[pl.ds(0, 16)]
            lane = lax.iota(jnp.int32, 16)
            degv[pl.ds(0, 16)] = v - jnp.where(
                lane == 0, jnp.float32(EPAD), jnp.float32(0.0))

        def div_group(g, _):
            d16 = degv[pl.ds(g * 16, 16)]
            r16 = 1.0 / jnp.maximum(d16, jnp.float32(1.0))
            for l in range(16):
                r = r16[l]
                i = g * 16 + l
                for k in range(HD // 16):
                    rows[i, pl.ds(k * 16, 16)] = (
                        rows[i, pl.ds(k * 16, 16)] * r)
            return 0
        lax.fori_loop(0, nrows // 16, div_group, 0)
        pltpu.sync_copy(rows.at[pl.ds(0, nrows)],
                        out_hbm.at[h, pl.ds(r0, nrows)])

    for c in range(NPT // CH):
        r0 = s * NPT + c * CH

        @pl.when(r0 + CH <= N)
        def _():
            fin_chunk(r0, CH)

        if LAST_SZ:
            @pl.when(r0 == LAST_R0)
            def _():
                fin_chunk(r0, LAST_SZ)


@jax.jit
def kernel(author_emb, edge_index):
    src = edge_index[0]
    dst = edge_index[1]

    author_pad = jnp.concatenate(
        [author_emb, jnp.zeros((16, D), author_emb.dtype)], axis=0)
    author_h = author_pad.reshape(NTAB, NC, HD).swapaxes(0, 1)

    srcp = jnp.concatenate(
        [src, jnp.full((EPAD,), N, jnp.int32)]).reshape(NS, NCHUNK, EC)
    dstp = jnp.concatenate(
        [dst, jnp.zeros((EPAD,), jnp.int32)]).reshape(NS, NCHUNK, EC)

    mesh = plsc.VectorSubcoreMesh(
        core_axis_name="c", subcore_axis_name="s",
        num_cores=NC, num_subcores=NS)

    out2 = pl.kernel(
        _tile_body,
        out_type=jax.ShapeDtypeStruct((NC, N, HD), jnp.float32),
        mesh=mesh,
        compiler_params=pltpu.CompilerParams(use_tc_tiling_on_sc=False),
        scratch_types=[
            pltpu.VMEM((NCHUNK, EC), jnp.int32),
            pltpu.VMEM((NCHUNK, EC), jnp.int32),
            pltpu.VMEM((CH, HD), jnp.float32),
            pltpu.VMEM((CH,), jnp.float32),
            pltpu.VMEM((NPT,), jnp.float32),
            pltpu.VMEM((CH,), jnp.float32),
            pltpu.VMEM_SHARED((NTAB, HD), jnp.float32),
            pltpu.VMEM_SHARED((NPADTOT, HD), jnp.float32),
            pltpu.VMEM_SHARED((NPADTOT,), jnp.float32),
            pltpu.SemaphoreType.DMA,
            pltpu.SemaphoreType.DMA,
        ],
    )(author_h, srcp, dstp)

    return jnp.concatenate([out2[0], out2[1]], axis=1)


# speedup vs baseline: 1.0266x; 1.0266x over previous
import jax
import jax.numpy as jnp
from jax import lax
from jax.experimental import pallas as pl
from jax.experimental.pallas import tpu as pltpu
from jax.experimental.pallas import tpu_sc as plsc

N = 10000
E = 320000
D = 128
HD = D // 2
NS = 16
NC = 2
CH = 128
NCHUNK = -(-E // (NS * CH))
EP = NCHUNK * NS * CH
EPAD = EP - E
NTAB = N + 16
TPT = NTAB // NS
NPT = 640
NPADTOT = NPT * NS
LAST_R0 = (N // CH) * CH
LAST_SZ = N - LAST_R0


def _tile_body(author_hbm, srcp_hbm, dstp_hbm, out_hbm,
               srcv, dstv, rows, ones, zbuf, degv, tab, acc, deg,
               gsem, dsem):
    h = lax.axis_index("c")
    s = lax.axis_index("s")

    def fill_rows(i, _):
        for k in range(HD // 16):
            rows[i, pl.ds(k * 16, 16)] = jnp.zeros((16,), jnp.float32)
        return 0
    lax.fori_loop(0, CH, fill_rows, 0)

    def fill_1d(i, _):
        zbuf[pl.ds(i * 16, 16)] = jnp.zeros((16,), jnp.float32)
        ones[pl.ds(i * 16, 16)] = jnp.ones((16,), jnp.float32)
        return 0
    lax.fori_loop(0, CH // 16, fill_1d, 0)

    def fill_z(i, _):
        zbuf[pl.ds(CH + i * 16, 16)] = jnp.zeros((16,), jnp.float32)
        return 0
    lax.fori_loop(0, (NPT - CH) // 16, fill_z, 0)

    n0 = s * NPT
    for c in range(NPT // CH):
        pltpu.sync_copy(rows, acc.at[pl.ds(n0 + c * CH, CH)])
    pltpu.sync_copy(zbuf, deg.at[pl.ds(n0, NPT)])
    pltpu.sync_copy(author_hbm.at[h, pl.ds(s * TPT, TPT)],
                    tab.at[pl.ds(s * TPT, TPT)])

    pltpu.sync_copy(srcp_hbm.at[s], srcv)
    pltpu.sync_copy(dstp_hbm.at[s], dstv)

    plsc.subcore_barrier()

    def chunk(j, _):
        pltpu.async_copy(tab.at[srcv.at[j]], rows, gsem).wait()
        pltpu.async_copy(ones, deg.at[dstv.at[j]], dsem, add=True)
        pltpu.sync_copy(rows, acc.at[dstv.at[j]], add=True)
        return 0
    lax.fori_loop(0, NCHUNK, chunk, 0)

    def drain(j, _):
        pltpu.make_async_copy(ones, deg.at[dstv.at[j]], dsem).wait()
        return 0
    lax.fori_loop(0, NCHUNK, drain, 0)

    plsc.subcore_barrier()

    def fin_chunk(r0, nrows):
        pltpu.sync_copy(acc.at[pl.ds(r0, nrows)], rows.at[pl.ds(0, nrows)])
        pltpu.sync_copy(deg.at[pl.ds(r0, nrows)], degv.at[pl.ds(0, nrows)])

        @pl.when(r0 == 0)
        def _():
            v = degv[pl.ds(0, 16)]
            lane = lax.iota(jnp.int32, 16)
            degv[pl.ds(0, 16)] = v - jnp.where(
                lane == 0, jnp.float32(EPAD), jnp.float32(0.0))

        def div_group(g, _):
            d16 = degv[pl.ds(g * 16, 16)]
            r16 = 1.0 / jnp.maximum(d16, jnp.float32(1.0))
            for l in range(16):
                r = r16[l]
                i = g * 16 + l
                for k in range(HD // 16):
                    rows[i, pl.ds(k * 16, 16)] = (
                        rows[i, pl.ds(k * 16, 16)] * r)
            return 0
        lax.fori_loop(0, nrows // 16, div_group, 0)
        pltpu.sync_copy(rows.at[pl.ds(0, nrows)],
                        out_hbm.at[h, pl.ds(r0, nrows)])

    for c in range(NPT // CH):
        r0 = s * NPT + c * CH

        @pl.when(r0 + CH <= N)
        def _():
            fin_chunk(r0, CH)

        if LAST_SZ:
            @pl.when(r0 == LAST_R0)
            def _():
                fin_chunk(r0, LAST_SZ)


@jax.jit
def kernel(author_emb, edge_index):
    src = edge_index[0]
    dst = edge_index[1]

    author_pad = jnp.concatenate(
        [author_emb, jnp.zeros((16, D), author_emb.dtype)], axis=0)
    author_h = author_pad.reshape(NTAB, NC, HD).swapaxes(0, 1)

    srcp = jnp.concatenate(
        [src, jnp.full((EPAD,), N, jnp.int32)]).reshape(NS, NCHUNK, CH)
    dstp = jnp.concatenate(
        [dst, jnp.zeros((EPAD,), jnp.int32)]).reshape(NS, NCHUNK, CH)

    mesh = plsc.VectorSubcoreMesh(
        core_axis_name="c", subcore_axis_name="s",
        num_cores=NC, num_subcores=NS)

    out2 = pl.kernel(
        _tile_body,
        out_type=jax.ShapeDtypeStruct((NC, N, HD), jnp.float32),
        mesh=mesh,
        compiler_params=pltpu.CompilerParams(use_tc_tiling_on_sc=False),
        scratch_types=[
            pltpu.VMEM((NCHUNK, CH), jnp.int32),
            pltpu.VMEM((NCHUNK, CH), jnp.int32),
            pltpu.VMEM((CH, HD), jnp.float32),
            pltpu.VMEM((CH,), jnp.float32),
            pltpu.VMEM((NPT,), jnp.float32),
            pltpu.VMEM((CH,), jnp.float32),
            pltpu.VMEM_SHARED((NTAB, HD), jnp.float32),
            pltpu.VMEM_SHARED((NPADTOT, HD), jnp.float32),
            pltpu.VMEM_SHARED((NPADTOT,), jnp.float32),
            pltpu.SemaphoreType.DMA,
            pltpu.SemaphoreType.DMA,
        ],
    )(author_h, srcp, dstp)

    return jnp.concatenate([out2[0], out2[1]], axis=1)
